# Initial kernel scaffold; baseline (speedup 1.0000x reference)
import jax
import jax.numpy as jnp
from jax.experimental import pallas as pl


def kernel(x, W_proj, b_proj, ln1_g, ln1_b, Wq, Wk, Wv, Wo, ln2_g, ln2_b, W1, b1, W2, b2, Wr1, br1, Wr2, br2, Wh1, bh1, Wh2, bh2):
    raise NotImplementedError("write your pallas kernel here")


# speedup vs baseline: 4.6400x; 4.6400x over previous
import jax
import jax.numpy as jnp
import numpy as np
from jax.experimental import pallas as pl
from jax.experimental.pallas import tpu as pltpu

_K = 4
_DEPTH = 2
_HEADS = 8
_DIM_HEAD = 64
_INNER = _HEADS * _DIM_HEAD
_DIM = 512
_MLP_DIM = 512
_INPUT_DIM = 2048
_NUM_CLASSES = 10
_K_ACTIVE = 2
_B = 2
_N = 1024

_ROW_TILE = 256


def _proj_kernel(x_ref, W_ref, b_ref, out_ref):
    out_ref[...] = jnp.dot(x_ref[...], W_ref[...],
                           preferred_element_type=jnp.float32) + b_ref[...]


def _router_kernel(H_ref, Wr1_ref, br1_ref, Wr2_ref, br2_ref,
                   g_ref, w_ref, idx_ref):
    m0 = jnp.mean(H_ref[0:_N, :], axis=0, keepdims=True)
    m1 = jnp.mean(H_ref[_N:2 * _N, :], axis=0, keepdims=True)
    bag = jnp.concatenate([m0, m1], axis=0)
    r = jnp.maximum(jnp.dot(bag, Wr1_ref[...],
                            preferred_element_type=jnp.float32)
                    + br1_ref[...], 0.0)
    logits = jnp.dot(r, Wr2_ref[...],
                     preferred_element_type=jnp.float32) + br2_ref[...]
    mx = jnp.max(logits, axis=-1, keepdims=True)
    e = jnp.exp(logits - mx)
    g = e / jnp.sum(e, axis=-1, keepdims=True)
    g_ref[...] = g
    iota = jax.lax.broadcasted_iota(jnp.int32, (_B, _K), 1)
    v0 = jnp.max(g, axis=-1, keepdims=True)
    i0 = jnp.min(jnp.where(g == v0, iota, _K), axis=-1, keepdims=True)
    gm = jnp.where(iota == i0, -jnp.inf, g)
    v1 = jnp.max(gm, axis=-1, keepdims=True)
    i1 = jnp.min(jnp.where(gm == v1, iota, _K), axis=-1, keepdims=True)
    s = v0 + v1 + 1e-8
    w_ref[...] = jnp.concatenate([v0 / s, v1 / s], axis=-1)
    idx_ref[...] = jnp.concatenate([i0, i1], axis=-1)


def _layer_norm(h, g, b):
    mu = jnp.mean(h, axis=-1, keepdims=True)
    var = jnp.mean((h - mu) ** 2, axis=-1, keepdims=True)
    return (h - mu) * jax.lax.rsqrt(var + 1e-5) * g + b


def _expert_kernel(idx_ref, H_ref, ln1g_ref, ln1b_ref, Wq_ref, Wk_ref,
                   Wv_ref, Wo_ref, ln2g_ref, ln2b_ref, W1_ref, b1_ref,
                   W2_ref, b2_ref, out_ref, h_scr):
    l = pl.program_id(1)

    @pl.when(l == 0)
    def _():
        h_scr[...] = H_ref[0]

    h = h_scr[...]
    hn = _layer_norm(h, ln1g_ref[0, 0, 0], ln1b_ref[0, 0, 0])
    q = jnp.dot(hn, Wq_ref[0, 0], preferred_element_type=jnp.float32)
    k = jnp.dot(hn, Wk_ref[0, 0], preferred_element_type=jnp.float32)
    v = jnp.dot(hn, Wv_ref[0, 0], preferred_element_type=jnp.float32)
    scale = 1.0 / np.sqrt(_DIM_HEAD)
    outs = []
    for hd in range(_HEADS):
        sl = slice(hd * _DIM_HEAD, (hd + 1) * _DIM_HEAD)
        qh, kh, vh = q[:, sl], k[:, sl], v[:, sl]
        s = jax.lax.dot_general(qh, kh, (((1,), (1,)), ((), ())),
                                preferred_element_type=jnp.float32) * scale
        s = s - jnp.max(s, axis=-1, keepdims=True)
        e = jnp.exp(s)
        p = e / jnp.sum(e, axis=-1, keepdims=True)
        outs.append(jnp.dot(p, vh, preferred_element_type=jnp.float32))
    o = jnp.concatenate(outs, axis=-1)
    h = h + jnp.dot(o, Wo_ref[0, 0], preferred_element_type=jnp.float32)
    hn2 = _layer_norm(h, ln2g_ref[0, 0, 0], ln2b_ref[0, 0, 0])
    mlp = jnp.maximum(
        jnp.dot(hn2, W1_ref[0, 0], preferred_element_type=jnp.float32)
        + b1_ref[0, 0, 0], 0.0)
    h = h + jnp.dot(mlp, W2_ref[0, 0],
                    preferred_element_type=jnp.float32) + b2_ref[0, 0, 0]
    h_scr[...] = h

    @pl.when(l == _DEPTH - 1)
    def _():
        out_ref[0, 0] = jnp.mean(h, axis=0)


def _head_kernel(lat_ref, w_ref, Wh1_ref, bh1_ref, Wh2_ref, bh2_ref,
                 z_ref, logit_ref):
    w = w_ref[...]
    z = (lat_ref[:, 0:_DIM] * w[:, 0:1]
         + lat_ref[:, _DIM:2 * _DIM] * w[:, 1:2])
    z_ref[...] = z
    r = jnp.maximum(jnp.dot(z, Wh1_ref[...],
                            preferred_element_type=jnp.float32)
                    + bh1_ref[...], 0.0)
    logit_ref[...] = jnp.dot(r, Wh2_ref[...],
                             preferred_element_type=jnp.float32) + bh2_ref[...]


def kernel(x, W_proj, b_proj, ln1_g, ln1_b, Wq, Wk, Wv, Wo, ln2_g, ln2_b,
           W1, b1, W2, b2, Wr1, br1, Wr2, br2, Wh1, bh1, Wh2, bh2):
    x2 = x.reshape(_B * _N, _INPUT_DIM)
    n_tiles = (_B * _N) // _ROW_TILE

    H2 = pl.pallas_call(
        _proj_kernel,
        grid=(n_tiles,),
        in_specs=[
            pl.BlockSpec((_ROW_TILE, _INPUT_DIM), lambda i: (i, 0)),
            pl.BlockSpec((_INPUT_DIM, _DIM), lambda i: (0, 0)),
            pl.BlockSpec((1, _DIM), lambda i: (0, 0)),
        ],
        out_specs=pl.BlockSpec((_ROW_TILE, _DIM), lambda i: (i, 0)),
        out_shape=jax.ShapeDtypeStruct((_B * _N, _DIM), jnp.float32),
        compiler_params=pltpu.CompilerParams(
            dimension_semantics=("arbitrary",)),
    )(x2, W_proj, b_proj.reshape(1, _DIM))

    g_soft, weights, idx = pl.pallas_call(
        _router_kernel,
        in_specs=[
            pl.BlockSpec((_B * _N, _DIM), lambda: (0, 0)),
            pl.BlockSpec((_DIM, 256), lambda: (0, 0)),
            pl.BlockSpec((1, 256), lambda: (0, 0)),
            pl.BlockSpec((256, _K), lambda: (0, 0)),
            pl.BlockSpec((1, _K), lambda: (0, 0)),
        ],
        out_specs=[
            pl.BlockSpec((_B, _K), lambda: (0, 0)),
            pl.BlockSpec((_B, _K_ACTIVE), lambda: (0, 0)),
            pl.BlockSpec((_B, _K_ACTIVE), lambda: (0, 0)),
        ],
        out_shape=[
            jax.ShapeDtypeStruct((_B, _K), jnp.float32),
            jax.ShapeDtypeStruct((_B, _K_ACTIVE), jnp.float32),
            jax.ShapeDtypeStruct((_B, _K_ACTIVE), jnp.int32),
        ],
    )(H2, Wr1, br1.reshape(1, 256), Wr2, br2.reshape(1, _K))

    idx_flat = idx.reshape(_B * _K_ACTIVE)
    H3 = H2.reshape(_B, _N, _DIM)
    n_pairs = _B * _K_ACTIVE

    grid_spec = pltpu.PrefetchScalarGridSpec(
        num_scalar_prefetch=1,
        grid=(n_pairs, _DEPTH),
        in_specs=[
            pl.BlockSpec((1, _N, _DIM), lambda p, l, idx: (p // _K_ACTIVE, 0, 0)),
            pl.BlockSpec((1, 1, 1, _DIM), lambda p, l, idx: (idx[p], l, 0, 0)),
            pl.BlockSpec((1, 1, 1, _DIM), lambda p, l, idx: (idx[p], l, 0, 0)),
            pl.BlockSpec((1, 1, _DIM, _INNER), lambda p, l, idx: (idx[p], l, 0, 0)),
            pl.BlockSpec((1, 1, _DIM, _INNER), lambda p, l, idx: (idx[p], l, 0, 0)),
            pl.BlockSpec((1, 1, _DIM, _INNER), lambda p, l, idx: (idx[p], l, 0, 0)),
            pl.BlockSpec((1, 1, _INNER, _DIM), lambda p, l, idx: (idx[p], l, 0, 0)),
            pl.BlockSpec((1, 1, 1, _DIM), lambda p, l, idx: (idx[p], l, 0, 0)),
            pl.BlockSpec((1, 1, 1, _DIM), lambda p, l, idx: (idx[p], l, 0, 0)),
            pl.BlockSpec((1, 1, _DIM, _MLP_DIM), lambda p, l, idx: (idx[p], l, 0, 0)),
            pl.BlockSpec((1, 1, 1, _MLP_DIM), lambda p, l, idx: (idx[p], l, 0, 0)),
            pl.BlockSpec((1, 1, _MLP_DIM, _DIM), lambda p, l, idx: (idx[p], l, 0, 0)),
            pl.BlockSpec((1, 1, 1, _DIM), lambda p, l, idx: (idx[p], l, 0, 0)),
        ],
        out_specs=pl.BlockSpec((1, 1, _DIM), lambda p, l, idx: (p, 0, 0)),
        scratch_shapes=[pltpu.VMEM((_N, _DIM), jnp.float32)],
    )

    latents = pl.pallas_call(
        _expert_kernel,
        grid_spec=grid_spec,
        out_shape=jax.ShapeDtypeStruct((n_pairs, 1, _DIM), jnp.float32),
        compiler_params=pltpu.CompilerParams(
            dimension_semantics=("arbitrary", "arbitrary")),
    )(idx_flat, H3,
      ln1_g.reshape(_K, _DEPTH, 1, _DIM), ln1_b.reshape(_K, _DEPTH, 1, _DIM),
      Wq, Wk, Wv, Wo,
      ln2_g.reshape(_K, _DEPTH, 1, _DIM), ln2_b.reshape(_K, _DEPTH, 1, _DIM),
      W1, b1.reshape(_K, _DEPTH, 1, _MLP_DIM),
      W2, b2.reshape(_K, _DEPTH, 1, _DIM))

    lat2 = latents.reshape(_B, _K_ACTIVE * _DIM)

    z, logits = pl.pallas_call(
        _head_kernel,
        in_specs=[
            pl.BlockSpec((_B, _K_ACTIVE * _DIM), lambda: (0, 0)),
            pl.BlockSpec((_B, _K_ACTIVE), lambda: (0, 0)),
            pl.BlockSpec((_DIM, 128), lambda: (0, 0)),
            pl.BlockSpec((1, 128), lambda: (0, 0)),
            pl.BlockSpec((128, _NUM_CLASSES), lambda: (0, 0)),
            pl.BlockSpec((1, _NUM_CLASSES), lambda: (0, 0)),
        ],
        out_specs=[
            pl.BlockSpec((_B, _DIM), lambda: (0, 0)),
            pl.BlockSpec((_B, _NUM_CLASSES), lambda: (0, 0)),
        ],
        out_shape=[
            jax.ShapeDtypeStruct((_B, _DIM), jnp.float32),
            jax.ShapeDtypeStruct((_B, _NUM_CLASSES), jnp.float32),
        ],
    )(lat2, weights, Wh1, bh1.reshape(1, 128), Wh2,
      bh2.reshape(1, _NUM_CLASSES))

    return (z, logits, g_soft)


# speedup vs baseline: 5.7889x; 1.2476x over previous
import jax
import jax.numpy as jnp
import numpy as np
from jax.experimental import pallas as pl
from jax.experimental.pallas import tpu as pltpu

_K = 4
_DEPTH = 2
_HEADS = 8
_DIM_HEAD = 64
_INNER = _HEADS * _DIM_HEAD
_DIM = 512
_MLP_DIM = 512
_INPUT_DIM = 2048
_NUM_CLASSES = 10
_K_ACTIVE = 2
_B = 2
_N = 1024

_ROW_TILE = 256
_TILES_PER_BAG = _N // _ROW_TILE
_N_TILES = (_B * _N) // _ROW_TILE


def _proj_router_kernel(x_ref, W_ref, b_ref, Wr1_ref, br1_ref, Wr2_ref,
                        br2_ref, H_ref, g_ref, w_ref, idx_ref, bag_scr):
    i = pl.program_id(0)
    Hblk = jnp.dot(x_ref[...], W_ref[...],
                   preferred_element_type=jnp.float32) + b_ref[...]
    H_ref[...] = Hblk
    part = jnp.sum(Hblk, axis=0, keepdims=True)
    bag = i // _TILES_PER_BAG

    @pl.when(i % _TILES_PER_BAG == 0)
    def _():
        bag_scr[pl.ds(bag, 1), :] = part

    @pl.when(i % _TILES_PER_BAG != 0)
    def _():
        bag_scr[pl.ds(bag, 1), :] = bag_scr[pl.ds(bag, 1), :] + part

    @pl.when(i == _N_TILES - 1)
    def _():
        bagm = bag_scr[0:_B, :] * (1.0 / _N)
        r = jnp.maximum(jnp.dot(bagm, Wr1_ref[...],
                                preferred_element_type=jnp.float32)
                        + br1_ref[...], 0.0)
        logits = jnp.dot(r, Wr2_ref[...],
                         preferred_element_type=jnp.float32) + br2_ref[...]
        mx = jnp.max(logits, axis=-1, keepdims=True)
        e = jnp.exp(logits - mx)
        g = e / jnp.sum(e, axis=-1, keepdims=True)
        g_ref[...] = g
        iota = jax.lax.broadcasted_iota(jnp.int32, (_B, _K), 1)
        v0 = jnp.max(g, axis=-1, keepdims=True)
        i0 = jnp.min(jnp.where(g == v0, iota, _K), axis=-1, keepdims=True)
        gm = jnp.where(iota == i0, -jnp.inf, g)
        v1 = jnp.max(gm, axis=-1, keepdims=True)
        i1 = jnp.min(jnp.where(gm == v1, iota, _K), axis=-1, keepdims=True)
        s = v0 + v1 + 1e-8
        w_ref[...] = jnp.concatenate([v0 / s, v1 / s], axis=-1)
        idx_ref[...] = jnp.concatenate([i0, i1], axis=-1)


def _layer_norm(h, g, b):
    mu = jnp.mean(h, axis=-1, keepdims=True)
    var = jnp.mean((h - mu) ** 2, axis=-1, keepdims=True)
    return (h - mu) * jax.lax.rsqrt(var + 1e-5) * g + b


def _expert_kernel(idx_ref, w_ref, H_ref, ln1g_ref, ln1b_ref, Wq_ref, Wk_ref,
                   Wv_ref, Wo_ref, ln2g_ref, ln2b_ref, W1_ref, b1_ref,
                   W2_ref, b2_ref, Wh1_ref, bh1_ref, Wh2_ref, bh2_ref,
                   z_ref, logit_ref, h_scr, o_scr, z_scr):
    p = pl.program_id(0)
    l = pl.program_id(1)

    @pl.when(l == 0)
    def _():
        h_scr[...] = H_ref[0]

    h = h_scr[...]
    hn = _layer_norm(h, ln1g_ref[0, 0, 0], ln1b_ref[0, 0, 0]
                     ).astype(jnp.bfloat16)
    scale = 1.0 / np.sqrt(_DIM_HEAD)
    bf = jnp.bfloat16
    q = (jnp.dot(hn, Wq_ref[0, 0].astype(bf),
                 preferred_element_type=jnp.float32) * scale).astype(bf)
    k = jnp.dot(hn, Wk_ref[0, 0].astype(bf),
                preferred_element_type=jnp.float32).astype(bf)
    v = jnp.dot(hn, Wv_ref[0, 0].astype(bf),
                preferred_element_type=jnp.float32).astype(bf)
    for hd in range(_HEADS):
        sl = slice(hd * _DIM_HEAD, (hd + 1) * _DIM_HEAD)
        qh, kh, vh = q[:, sl], k[:, sl], v[:, sl]
        s = jax.lax.dot_general(qh, kh, (((1,), (1,)), ((), ())),
                                preferred_element_type=jnp.float32)
        e = jnp.exp(s)
        eb = e.astype(bf)
        num = jnp.dot(eb, vh, preferred_element_type=jnp.float32)
        den = jnp.sum(e, axis=-1, keepdims=True)
        o_scr[:, sl] = (num / den).astype(bf)
    h = h + jnp.dot(o_scr[...], Wo_ref[0, 0].astype(bf),
                    preferred_element_type=jnp.float32)
    hn2 = _layer_norm(h, ln2g_ref[0, 0, 0], ln2b_ref[0, 0, 0]).astype(bf)
    mlp = jnp.maximum(
        jnp.dot(hn2, W1_ref[0, 0].astype(bf),
                preferred_element_type=jnp.float32)
        + b1_ref[0, 0, 0], 0.0).astype(bf)
    h = h + jnp.dot(mlp, W2_ref[0, 0].astype(bf),
                    preferred_element_type=jnp.float32) + b2_ref[0, 0, 0]
    h_scr[...] = h

    @pl.when(l == _DEPTH - 1)
    def _():
        lat = jnp.mean(h, axis=0, keepdims=True)
        wv = w_ref[p]
        bag = p // _K_ACTIVE
        contrib = lat * wv

        @pl.when(p % _K_ACTIVE == 0)
        def _():
            z_scr[pl.ds(bag, 1), :] = contrib

        @pl.when(p % _K_ACTIVE != 0)
        def _():
            z_scr[pl.ds(bag, 1), :] = z_scr[pl.ds(bag, 1), :] + contrib

        @pl.when(p == _B * _K_ACTIVE - 1)
        def _():
            z = z_scr[0:_B, :]
            z_ref[...] = z
            r = jnp.maximum(jnp.dot(z, Wh1_ref[...],
                                    preferred_element_type=jnp.float32)
                            + bh1_ref[...], 0.0)
            logit_ref[...] = jnp.dot(r, Wh2_ref[...],
                                     preferred_element_type=jnp.float32
                                     ) + bh2_ref[...]


def kernel(x, W_proj, b_proj, ln1_g, ln1_b, Wq, Wk, Wv, Wo, ln2_g, ln2_b,
           W1, b1, W2, b2, Wr1, br1, Wr2, br2, Wh1, bh1, Wh2, bh2):
    x2 = x.reshape(_B * _N, _INPUT_DIM)

    H2, g_soft, weights, idx = pl.pallas_call(
        _proj_router_kernel,
        grid=(_N_TILES,),
        in_specs=[
            pl.BlockSpec((_ROW_TILE, _INPUT_DIM), lambda i: (i, 0)),
            pl.BlockSpec((_INPUT_DIM, _DIM), lambda i: (0, 0)),
            pl.BlockSpec((1, _DIM), lambda i: (0, 0)),
            pl.BlockSpec((_DIM, 256), lambda i: (0, 0)),
            pl.BlockSpec((1, 256), lambda i: (0, 0)),
            pl.BlockSpec((256, _K), lambda i: (0, 0)),
            pl.BlockSpec((1, _K), lambda i: (0, 0)),
        ],
        out_specs=[
            pl.BlockSpec((_ROW_TILE, _DIM), lambda i: (i, 0)),
            pl.BlockSpec((_B, _K), lambda i: (0, 0)),
            pl.BlockSpec((_B, _K_ACTIVE), lambda i: (0, 0)),
            pl.BlockSpec((_B, _K_ACTIVE), lambda i: (0, 0)),
        ],
        out_shape=[
            jax.ShapeDtypeStruct((_B * _N, _DIM), jnp.float32),
            jax.ShapeDtypeStruct((_B, _K), jnp.float32),
            jax.ShapeDtypeStruct((_B, _K_ACTIVE), jnp.float32),
            jax.ShapeDtypeStruct((_B, _K_ACTIVE), jnp.int32),
        ],
        scratch_shapes=[pltpu.VMEM((8, _DIM), jnp.float32)],
        compiler_params=pltpu.CompilerParams(
            dimension_semantics=("arbitrary",)),
    )(x2, W_proj, b_proj.reshape(1, _DIM), Wr1, br1.reshape(1, 256),
      Wr2, br2.reshape(1, _K))

    idx_flat = idx.reshape(_B * _K_ACTIVE)
    w_flat = weights.reshape(_B * _K_ACTIVE)
    H3 = H2.reshape(_B, _N, _DIM)
    n_pairs = _B * _K_ACTIVE

    grid_spec = pltpu.PrefetchScalarGridSpec(
        num_scalar_prefetch=2,
        grid=(n_pairs, _DEPTH),
        in_specs=[
            pl.BlockSpec((1, _N, _DIM),
                         lambda p, l, idx, w: (p // _K_ACTIVE, 0, 0)),
            pl.BlockSpec((1, 1, 1, _DIM), lambda p, l, idx, w: (idx[p], l, 0, 0)),
            pl.BlockSpec((1, 1, 1, _DIM), lambda p, l, idx, w: (idx[p], l, 0, 0)),
            pl.BlockSpec((1, 1, _DIM, _INNER), lambda p, l, idx, w: (idx[p], l, 0, 0)),
            pl.BlockSpec((1, 1, _DIM, _INNER), lambda p, l, idx, w: (idx[p], l, 0, 0)),
            pl.BlockSpec((1, 1, _DIM, _INNER), lambda p, l, idx, w: (idx[p], l, 0, 0)),
            pl.BlockSpec((1, 1, _INNER, _DIM), lambda p, l, idx, w: (idx[p], l, 0, 0)),
            pl.BlockSpec((1, 1, 1, _DIM), lambda p, l, idx, w: (idx[p], l, 0, 0)),
            pl.BlockSpec((1, 1, 1, _DIM), lambda p, l, idx, w: (idx[p], l, 0, 0)),
            pl.BlockSpec((1, 1, _DIM, _MLP_DIM), lambda p, l, idx, w: (idx[p], l, 0, 0)),
            pl.BlockSpec((1, 1, 1, _MLP_DIM), lambda p, l, idx, w: (idx[p], l, 0, 0)),
            pl.BlockSpec((1, 1, _MLP_DIM, _DIM), lambda p, l, idx, w: (idx[p], l, 0, 0)),
            pl.BlockSpec((1, 1, 1, _DIM), lambda p, l, idx, w: (idx[p], l, 0, 0)),
            pl.BlockSpec((_DIM, 128), lambda p, l, idx, w: (0, 0)),
            pl.BlockSpec((1, 128), lambda p, l, idx, w: (0, 0)),
            pl.BlockSpec((128, _NUM_CLASSES), lambda p, l, idx, w: (0, 0)),
            pl.BlockSpec((1, _NUM_CLASSES), lambda p, l, idx, w: (0, 0)),
        ],
        out_specs=[
            pl.BlockSpec((_B, _DIM), lambda p, l, idx, w: (0, 0)),
            pl.BlockSpec((_B, _NUM_CLASSES), lambda p, l, idx, w: (0, 0)),
        ],
        scratch_shapes=[pltpu.VMEM((_N, _DIM), jnp.float32),
                        pltpu.VMEM((_N, _INNER), jnp.bfloat16),
                        pltpu.VMEM((8, _DIM), jnp.float32)],
    )

    z, logits = pl.pallas_call(
        _expert_kernel,
        grid_spec=grid_spec,
        out_shape=[
            jax.ShapeDtypeStruct((_B, _DIM), jnp.float32),
            jax.ShapeDtypeStruct((_B, _NUM_CLASSES), jnp.float32),
        ],
        compiler_params=pltpu.CompilerParams(
            dimension_semantics=("arbitrary", "arbitrary")),
    )(idx_flat, w_flat, H3,
      ln1_g.reshape(_K, _DEPTH, 1, _DIM), ln1_b.reshape(_K, _DEPTH, 1, _DIM),
      Wq, Wk, Wv, Wo,
      ln2_g.reshape(_K, _DEPTH, 1, _DIM), ln2_b.reshape(_K, _DEPTH, 1, _DIM),
      W1, b1.reshape(_K, _DEPTH, 1, _MLP_DIM),
      W2, b2.reshape(_K, _DEPTH, 1, _DIM),
      Wh1, bh1.reshape(1, 128), Wh2, bh2.reshape(1, _NUM_CLASSES))

    return (z, logits, g_soft)
